# baseline (speedup 1.0000x reference)
import functools

import jax
import jax.numpy as jnp
from jax import lax
from jax.experimental import pallas as pl
from jax.experimental.pallas import tpu as pltpu
from jax.experimental.pallas import tpu_sc as plsc

_VOCAB = 1000000
_HIDDEN = 64
_BATCH = 4096
_SEQ = 50
_PAD = 0
_EPS = 1e-8

_NC = 2
_NS = 16
_NW = _NC * _NS
_ROWS = _BATCH * _SEQ
_RPW = _ROWS // _NW
_PIECE = 128
_NPIECE = _RPW // _PIECE
_NBUF = 10
_GLEAD = 8

_INV_H = 1.0 / _HIDDEN


def _rsqrt(x):
    i = lax.bitcast_convert_type(x, jnp.int32)
    i = jnp.int32(0x5F3759DF) - lax.shift_right_logical(i, 1)
    y = lax.bitcast_convert_type(i, jnp.float32)
    half = x * 0.5
    y = y * (1.5 - half * y * y)
    y = y * (1.5 - half * y * y)
    y = y * (1.5 - half * y * y)
    return y


def _sc_body(tok_hbm, words_hbm, pos_hbm, wb_hbm, out_hbm,
             idx_v, pos_v, wb_v, gsems, osems, *bufs):
    wid = lax.axis_index("s") * _NC + lax.axis_index("c")
    base = wid * _RPW

    pltpu.sync_copy(tok_hbm.at[pl.ds(base, _RPW)], idx_v.at[pl.ds(0, _RPW)])
    pltpu.sync_copy(pos_hbm, pos_v)
    pltpu.sync_copy(wb_hbm, wb_v)

    def start_gather(p, buf, sem):
        pltpu.async_copy(words_hbm.at[idx_v.at[pl.ds(p * _PIECE, _PIECE)]],
                         buf, sem)

    def wait_gather(p, buf, sem):
        pltpu.make_async_copy(
            words_hbm.at[idx_v.at[pl.ds(p * _PIECE, _PIECE)]], buf, sem
        ).wait()

    def start_out(p, buf, sem):
        pltpu.async_copy(buf, out_hbm.at[pl.ds(base + p * _PIECE, _PIECE)],
                         sem)

    def wait_out(p, buf, sem):
        pltpu.make_async_copy(
            buf, out_hbm.at[pl.ds(base + p * _PIECE, _PIECE)], sem
        ).wait()

    w0 = wb_v[pl.ds(0, 16)]
    w1 = wb_v[pl.ds(16, 16)]
    w2 = wb_v[pl.ds(32, 16)]
    w3 = wb_v[pl.ds(48, 16)]
    b0 = wb_v[pl.ds(64, 16)]
    b1 = wb_v[pl.ds(80, 16)]
    b2 = wb_v[pl.ds(96, 16)]
    b3 = wb_v[pl.ds(112, 16)]

    def compute(p, rb):
        prow = p * _PIECE

        @plsc.parallel_loop(0, _PIECE, unroll=4)
        def row_body(r):
            row = prow + r
            tok = idx_v[pl.ds(row, 16)][0]
            pb = lax.rem(row, _SEQ) * _HIDDEN
            x0 = rb[r, pl.ds(0, 16)] + pos_v[pl.ds(pb, 16)]
            x1 = rb[r, pl.ds(16, 16)] + pos_v[pl.ds(pb + 16, 16)]
            x2 = rb[r, pl.ds(32, 16)] + pos_v[pl.ds(pb + 32, 16)]
            x3 = rb[r, pl.ds(48, 16)] + pos_v[pl.ds(pb + 48, 16)]
            s = (x0 + x1) + (x2 + x3)
            sq = (x0 * x0 + x1 * x1) + (x2 * x2 + x3 * x3)
            mean = plsc.cumsum(s)[15] * _INV_H
            ex2 = plsc.cumsum(sq)[15] * _INV_H
            var = ex2 - mean * mean
            rstd = _rsqrt(var + _EPS)
            scale = jnp.where(tok != _PAD, rstd, 0.0)
            shift = -mean * scale
            live = jnp.where(tok != _PAD, 1.0, 0.0)
            rb[r, pl.ds(0, 16)] = (x0 * scale + shift) * w0 + b0 * live
            rb[r, pl.ds(16, 16)] = (x1 * scale + shift) * w1 + b1 * live
            rb[r, pl.ds(32, 16)] = (x2 * scale + shift) * w2 + b2 * live
            rb[r, pl.ds(48, 16)] = (x3 * scale + shift) * w3 + b3 * live

    for j in range(_GLEAD):
        start_gather(j, bufs[j], gsems[j])

    def step(p, j):
        b = j
        wait_gather(p, bufs[b], gsems[b])
        compute(p, bufs[b])
        start_out(p, bufs[b], osems[b])
        nb = (j + _GLEAD) % _NBUF

        @pl.when(p >= _NBUF - _GLEAD)
        def _():
            wait_out(p - (_NBUF - _GLEAD), bufs[nb], osems[nb])

        @pl.when(p + _GLEAD < _NPIECE)
        def _():
            start_gather(p + _GLEAD, bufs[nb], gsems[nb])

    def main_body(g, _):
        for j in range(_NBUF):
            step(g + j, j)
        return ()

    n_main = _NPIECE // _NBUF * _NBUF
    lax.fori_loop(0, n_main // _NBUF, lambda i, c: main_body(i * _NBUF, c),
                  ())
    for p in range(n_main, _NPIECE):
        step(p, p % _NBUF)

    for p in range(_NPIECE - (_NBUF - _GLEAD), _NPIECE):
        wait_out(p, bufs[p % _NBUF], osems[p % _NBUF])


_TW = 1664


_NTB = (_VOCAB + _TW - 1) // _TW


def _tc_transpose(wT):
    def body(in_ref, out_ref):
        r = jax.lax.broadcasted_iota(jnp.int32, (_HIDDEN, _HIDDEN), 0)
        c = jax.lax.broadcasted_iota(jnp.int32, (_HIDDEN, _HIDDEN), 1)
        eye = (r == c).astype(jnp.float32)
        x = in_ref[...]
        xt = jax.lax.dot_general(x, eye, (((0,), (0,)), ((), ())),
                                 preferred_element_type=jnp.float32)
        lo = xt[: _TW // 2, :]
        hi = xt[_TW // 2 :, :]
        out_ref[...] = jnp.concatenate([lo, hi], axis=1)

    return pl.pallas_call(
        body,
        grid=(_NTB,),
        in_specs=[pl.BlockSpec((_HIDDEN, _TW), lambda i: (0, i))],
        out_specs=pl.BlockSpec((_TW // 2, 2 * _HIDDEN), lambda i: (i, 0)),
        out_shape=jax.ShapeDtypeStruct((_NTB * _TW // 2, 2 * _HIDDEN),
                                       jnp.float32),
    )(wT)


@jax.jit
def _sc_call(toks, words, pos_flat, wb):
    mesh = plsc.VectorSubcoreMesh(core_axis_name="c", subcore_axis_name="s",
                                  num_cores=_NC, num_subcores=_NS)
    f = pl.kernel(
        _sc_body,
        out_type=jax.ShapeDtypeStruct((_ROWS, _HIDDEN), jnp.float32),
        mesh=mesh,
        compiler_params=pltpu.CompilerParams(needs_layout_passes=False,
                                             use_tc_tiling_on_sc=False),
        scratch_types=[
            pltpu.VMEM((_RPW + 16,), jnp.int32),
            pltpu.VMEM((_SEQ * _HIDDEN,), jnp.float32),
            pltpu.VMEM((2 * _HIDDEN,), jnp.float32),
            [pltpu.SemaphoreType.DMA] * _NBUF,
            [pltpu.SemaphoreType.DMA] * _NBUF,
        ] + [pltpu.VMEM((_PIECE, _HIDDEN), jnp.float32)] * _NBUF,
    )
    return f(toks, words, pos_flat, wb)


def kernel(tokens, words, pos_emb, ln_weight, ln_bias):
    toks = tokens.reshape(-1).astype(jnp.int32)
    tb = toks // _TW
    tr = toks % _TW
    half = _TW // 2
    toks = tb * _TW + jnp.where(tr < half, tr * 2, (tr - half) * 2 + 1)
    wlin = _tc_transpose(words.T).reshape(_NTB * _TW, _HIDDEN)
    pos_flat = pos_emb.reshape(-1).astype(jnp.float32)
    wb = jnp.concatenate([ln_weight, ln_bias]).astype(jnp.float32)
    out = _sc_call(toks, wlin, pos_flat, wb)
    return out.reshape(tokens.shape + (_HIDDEN,))


# speedup vs baseline: 1.0187x; 1.0187x over previous
import functools

import jax
import jax.numpy as jnp
from jax import lax
from jax.experimental import pallas as pl
from jax.experimental.pallas import tpu as pltpu
from jax.experimental.pallas import tpu_sc as plsc

_VOCAB = 1000000
_HIDDEN = 64
_BATCH = 4096
_SEQ = 50
_PAD = 0
_EPS = 1e-8

_NC = 2
_NS = 16
_NW = _NC * _NS
_ROWS = _BATCH * _SEQ
_RPW = _ROWS // _NW
_PIECE = 128
_NPIECE = _RPW // _PIECE
_NBUF = 10
_GLEAD = 8

_INV_H = 1.0 / _HIDDEN


def _rsqrt(x):
    i = lax.bitcast_convert_type(x, jnp.int32)
    i = jnp.int32(0x5F3759DF) - lax.shift_right_logical(i, 1)
    y = lax.bitcast_convert_type(i, jnp.float32)
    half = x * 0.5
    y = y * (1.5 - half * y * y)
    y = y * (1.5 - half * y * y)
    y = y * (1.5 - half * y * y)
    return y


def _sc_body(tok_hbm, words_hbm, pos_hbm, wb_hbm, out_hbm,
             idx_v, pos_v, wb_v, gsems, osems, *bufs):
    wid = lax.axis_index("s") * _NC + lax.axis_index("c")
    base = wid * _RPW

    pltpu.sync_copy(tok_hbm.at[pl.ds(base, _RPW)], idx_v.at[pl.ds(0, _RPW)])
    pltpu.sync_copy(pos_hbm, pos_v)
    pltpu.sync_copy(wb_hbm, wb_v)

    def start_gather(p, buf, sem):
        pltpu.async_copy(words_hbm.at[idx_v.at[pl.ds(p * _PIECE, _PIECE)]],
                         buf, sem)

    def wait_gather(p, buf, sem):
        pltpu.make_async_copy(
            words_hbm.at[idx_v.at[pl.ds(p * _PIECE, _PIECE)]], buf, sem
        ).wait()

    def start_out(p, buf, sem):
        pltpu.async_copy(buf, out_hbm.at[pl.ds(base + p * _PIECE, _PIECE)],
                         sem)

    def wait_out(p, buf, sem):
        pltpu.make_async_copy(
            buf, out_hbm.at[pl.ds(base + p * _PIECE, _PIECE)], sem
        ).wait()

    w0 = wb_v[pl.ds(0, 16)]
    w1 = wb_v[pl.ds(16, 16)]
    w2 = wb_v[pl.ds(32, 16)]
    w3 = wb_v[pl.ds(48, 16)]
    b0 = wb_v[pl.ds(64, 16)]
    b1 = wb_v[pl.ds(80, 16)]
    b2 = wb_v[pl.ds(96, 16)]
    b3 = wb_v[pl.ds(112, 16)]

    def compute(p, rb):
        prow = p * _PIECE

        @plsc.parallel_loop(0, _PIECE, unroll=4)
        def row_body(r):
            row = prow + r
            tok = idx_v[pl.ds(row, 16)][0]
            pb = lax.rem(row, _SEQ) * _HIDDEN
            x0 = rb[r, pl.ds(0, 16)] + pos_v[pl.ds(pb, 16)]
            x1 = rb[r, pl.ds(16, 16)] + pos_v[pl.ds(pb + 16, 16)]
            x2 = rb[r, pl.ds(32, 16)] + pos_v[pl.ds(pb + 32, 16)]
            x3 = rb[r, pl.ds(48, 16)] + pos_v[pl.ds(pb + 48, 16)]
            s = (x0 + x1) + (x2 + x3)
            sq = (x0 * x0 + x1 * x1) + (x2 * x2 + x3 * x3)
            mean = plsc.cumsum(s)[15] * _INV_H
            ex2 = plsc.cumsum(sq)[15] * _INV_H
            var = ex2 - mean * mean
            rstd = _rsqrt(var + _EPS)
            scale = jnp.where(tok != _PAD, rstd, 0.0)
            shift = -mean * scale
            live = jnp.where(tok != _PAD, 1.0, 0.0)
            rb[r, pl.ds(0, 16)] = (x0 * scale + shift) * w0 + b0 * live
            rb[r, pl.ds(16, 16)] = (x1 * scale + shift) * w1 + b1 * live
            rb[r, pl.ds(32, 16)] = (x2 * scale + shift) * w2 + b2 * live
            rb[r, pl.ds(48, 16)] = (x3 * scale + shift) * w3 + b3 * live

    for j in range(_GLEAD):
        start_gather(j, bufs[j], gsems[j])

    def step(p, j):
        b = j
        wait_gather(p, bufs[b], gsems[b])
        compute(p, bufs[b])
        start_out(p, bufs[b], osems[b])
        nb = (j + _GLEAD) % _NBUF

        @pl.when(p >= _NBUF - _GLEAD)
        def _():
            wait_out(p - (_NBUF - _GLEAD), bufs[nb], osems[nb])

        @pl.when(p + _GLEAD < _NPIECE)
        def _():
            start_gather(p + _GLEAD, bufs[nb], gsems[nb])

    def main_body(g, _):
        for j in range(_NBUF):
            step(g + j, j)
        return ()

    n_main = _NPIECE // _NBUF * _NBUF
    lax.fori_loop(0, n_main // _NBUF, lambda i, c: main_body(i * _NBUF, c),
                  ())
    for p in range(n_main, _NPIECE):
        step(p, p % _NBUF)

    for p in range(_NPIECE - (_NBUF - _GLEAD), _NPIECE):
        wait_out(p, bufs[p % _NBUF], osems[p % _NBUF])


_TW = 1664


_NTB = (_VOCAB + _TW - 1) // _TW


def _tc_transpose(wT):
    def body(in_ref, out_ref):
        xt = in_ref[...].T
        out_ref[:, : _HIDDEN] = xt[: _TW // 2, :]
        out_ref[:, _HIDDEN :] = xt[_TW // 2 :, :]

    return pl.pallas_call(
        body,
        grid=(_NTB,),
        in_specs=[pl.BlockSpec((_HIDDEN, _TW), lambda i: (0, i))],
        out_specs=pl.BlockSpec((_TW // 2, 2 * _HIDDEN), lambda i: (i, 0)),
        out_shape=jax.ShapeDtypeStruct((_NTB * _TW // 2, 2 * _HIDDEN),
                                       jnp.float32),
    )(wT)


@jax.jit
def _sc_call(toks, words, pos_flat, wb):
    mesh = plsc.VectorSubcoreMesh(core_axis_name="c", subcore_axis_name="s",
                                  num_cores=_NC, num_subcores=_NS)
    f = pl.kernel(
        _sc_body,
        out_type=jax.ShapeDtypeStruct((_ROWS, _HIDDEN), jnp.float32),
        mesh=mesh,
        compiler_params=pltpu.CompilerParams(needs_layout_passes=False,
                                             use_tc_tiling_on_sc=False),
        scratch_types=[
            pltpu.VMEM((_RPW + 16,), jnp.int32),
            pltpu.VMEM((_SEQ * _HIDDEN,), jnp.float32),
            pltpu.VMEM((2 * _HIDDEN,), jnp.float32),
            [pltpu.SemaphoreType.DMA] * _NBUF,
            [pltpu.SemaphoreType.DMA] * _NBUF,
        ] + [pltpu.VMEM((_PIECE, _HIDDEN), jnp.float32)] * _NBUF,
    )
    return f(toks, words, pos_flat, wb)


def kernel(tokens, words, pos_emb, ln_weight, ln_bias):
    toks = tokens.reshape(-1).astype(jnp.int32)
    tb = toks // _TW
    tr = toks % _TW
    half = _TW // 2
    toks = tb * _TW + jnp.where(tr < half, tr * 2, (tr - half) * 2 + 1)
    wlin = _tc_transpose(words.T).reshape(_NTB * _TW, _HIDDEN)
    pos_flat = pos_emb.reshape(-1).astype(jnp.float32)
    wb = jnp.concatenate([ln_weight, ln_bias]).astype(jnp.float32)
    out = _sc_call(toks, wlin, pos_flat, wb)
    return out.reshape(tokens.shape + (_HIDDEN,))


# speedup vs baseline: 1.2511x; 1.2282x over previous
import functools

import jax
import jax.numpy as jnp
from jax import lax
from jax.experimental import pallas as pl
from jax.experimental.pallas import tpu as pltpu
from jax.experimental.pallas import tpu_sc as plsc

_VOCAB = 1000000
_HIDDEN = 64
_BATCH = 4096
_SEQ = 50
_PAD = 0
_EPS = 1e-8

_NC = 2
_NS = 16
_NW = _NC * _NS
_ROWS = _BATCH * _SEQ
_RPW = _ROWS // _NW
_PIECE = 128
_NPIECE = _RPW // _PIECE
_NBUF = 10
_GLEAD = 8

_INV_H = 1.0 / _HIDDEN


def _rsqrt(x):
    i = lax.bitcast_convert_type(x, jnp.int32)
    i = jnp.int32(0x5F3759DF) - lax.shift_right_logical(i, 1)
    y = lax.bitcast_convert_type(i, jnp.float32)
    half = x * 0.5
    y = y * (1.5 - half * y * y)
    y = y * (1.5 - half * y * y)
    y = y * (1.5 - half * y * y)
    return y


def _sc_body(tok_hbm, words_hbm, pos_hbm, wb_hbm, out_hbm,
             idx_v, pos_v, wb_v, gsems, osems, *bufs):
    wid = lax.axis_index("s") * _NC + lax.axis_index("c")
    base = wid * _RPW

    pltpu.sync_copy(tok_hbm.at[pl.ds(base, _RPW)], idx_v.at[pl.ds(0, _RPW)])
    pltpu.sync_copy(pos_hbm, pos_v)
    pltpu.sync_copy(wb_hbm, wb_v)

    def start_gather(p, buf, sem):
        pltpu.async_copy(words_hbm.at[idx_v.at[pl.ds(p * _PIECE, _PIECE)]],
                         buf, sem)

    def wait_gather(p, buf, sem):
        pltpu.make_async_copy(
            words_hbm.at[idx_v.at[pl.ds(p * _PIECE, _PIECE)]], buf, sem
        ).wait()

    def start_out(p, buf, sem):
        pltpu.async_copy(buf, out_hbm.at[pl.ds(base + p * _PIECE, _PIECE)],
                         sem)

    def wait_out(p, buf, sem):
        pltpu.make_async_copy(
            buf, out_hbm.at[pl.ds(base + p * _PIECE, _PIECE)], sem
        ).wait()

    w0 = wb_v[pl.ds(0, 16)]
    w1 = wb_v[pl.ds(16, 16)]
    w2 = wb_v[pl.ds(32, 16)]
    w3 = wb_v[pl.ds(48, 16)]
    b0 = wb_v[pl.ds(64, 16)]
    b1 = wb_v[pl.ds(80, 16)]
    b2 = wb_v[pl.ds(96, 16)]
    b3 = wb_v[pl.ds(112, 16)]

    def compute(p, rb):
        prow = p * _PIECE

        @plsc.parallel_loop(0, _PIECE, unroll=4)
        def row_body(r):
            row = prow + r
            tok = idx_v[pl.ds(row, 16)][0]
            pb = lax.rem(row, _SEQ) * _HIDDEN
            x0 = rb[r, pl.ds(0, 16)] + pos_v[pl.ds(pb, 16)]
            x1 = rb[r, pl.ds(16, 16)] + pos_v[pl.ds(pb + 16, 16)]
            x2 = rb[r, pl.ds(32, 16)] + pos_v[pl.ds(pb + 32, 16)]
            x3 = rb[r, pl.ds(48, 16)] + pos_v[pl.ds(pb + 48, 16)]
            s = (x0 + x1) + (x2 + x3)
            sq = (x0 * x0 + x1 * x1) + (x2 * x2 + x3 * x3)
            mean = plsc.cumsum(s)[15] * _INV_H
            ex2 = plsc.cumsum(sq)[15] * _INV_H
            var = ex2 - mean * mean
            rstd = _rsqrt(var + _EPS)
            scale = jnp.where(tok != _PAD, rstd, 0.0)
            shift = -mean * scale
            live = jnp.where(tok != _PAD, 1.0, 0.0)
            rb[r, pl.ds(0, 16)] = (x0 * scale + shift) * w0 + b0 * live
            rb[r, pl.ds(16, 16)] = (x1 * scale + shift) * w1 + b1 * live
            rb[r, pl.ds(32, 16)] = (x2 * scale + shift) * w2 + b2 * live
            rb[r, pl.ds(48, 16)] = (x3 * scale + shift) * w3 + b3 * live

    for j in range(_GLEAD):
        start_gather(j, bufs[j], gsems[j])

    def step(p, j):
        b = j
        wait_gather(p, bufs[b], gsems[b])
        compute(p, bufs[b])
        start_out(p, bufs[b], osems[b])
        nb = (j + _GLEAD) % _NBUF

        @pl.when(p >= _NBUF - _GLEAD)
        def _():
            wait_out(p - (_NBUF - _GLEAD), bufs[nb], osems[nb])

        @pl.when(p + _GLEAD < _NPIECE)
        def _():
            start_gather(p + _GLEAD, bufs[nb], gsems[nb])

    def main_body(g, _):
        for j in range(_NBUF):
            step(g + j, j)
        return ()

    n_main = _NPIECE // _NBUF * _NBUF
    lax.fori_loop(0, n_main // _NBUF, lambda i, c: main_body(i * _NBUF, c),
                  ())
    for p in range(n_main, _NPIECE):
        step(p, p % _NBUF)

    for p in range(_NPIECE - (_NBUF - _GLEAD), _NPIECE):
        wait_out(p, bufs[p % _NBUF], osems[p % _NBUF])


_TW = 3328


_NTB = (_VOCAB + _TW - 1) // _TW


def _tc_transpose(wT):
    def body(in_ref, out_ref):
        xt = in_ref[...].T
        out_ref[:, : _HIDDEN] = xt[: _TW // 2, :]
        out_ref[:, _HIDDEN :] = xt[_TW // 2 :, :]

    return pl.pallas_call(
        body,
        grid=(_NTB,),
        in_specs=[pl.BlockSpec((_HIDDEN, _TW), lambda i: (0, i))],
        out_specs=pl.BlockSpec((_TW // 2, 2 * _HIDDEN), lambda i: (i, 0)),
        out_shape=jax.ShapeDtypeStruct((_NTB * _TW // 2, 2 * _HIDDEN),
                                       jnp.float32),
    )(wT)


@jax.jit
def _sc_call(toks, words, pos_flat, wb):
    mesh = plsc.VectorSubcoreMesh(core_axis_name="c", subcore_axis_name="s",
                                  num_cores=_NC, num_subcores=_NS)
    f = pl.kernel(
        _sc_body,
        out_type=jax.ShapeDtypeStruct((_ROWS, _HIDDEN), jnp.float32),
        mesh=mesh,
        compiler_params=pltpu.CompilerParams(needs_layout_passes=False,
                                             use_tc_tiling_on_sc=False),
        scratch_types=[
            pltpu.VMEM((_RPW + 16,), jnp.int32),
            pltpu.VMEM((_SEQ * _HIDDEN,), jnp.float32),
            pltpu.VMEM((2 * _HIDDEN,), jnp.float32),
            [pltpu.SemaphoreType.DMA] * _NBUF,
            [pltpu.SemaphoreType.DMA] * _NBUF,
        ] + [pltpu.VMEM((_PIECE, _HIDDEN), jnp.float32)] * _NBUF,
    )
    return f(toks, words, pos_flat, wb)


def kernel(tokens, words, pos_emb, ln_weight, ln_bias):
    toks = tokens.reshape(-1).astype(jnp.int32)
    tb = toks // _TW
    tr = toks % _TW
    half = _TW // 2
    toks = tb * _TW + jnp.where(tr < half, tr * 2, (tr - half) * 2 + 1)
    wlin = _tc_transpose(words.T).reshape(_NTB * _TW, _HIDDEN)
    pos_flat = pos_emb.reshape(-1).astype(jnp.float32)
    wb = jnp.concatenate([ln_weight, ln_bias]).astype(jnp.float32)
    out = _sc_call(toks, wlin, pos_flat, wb)
    return out.reshape(tokens.shape + (_HIDDEN,))


# speedup vs baseline: 1.4300x; 1.1430x over previous
import functools

import jax
import jax.numpy as jnp
from jax import lax
from jax.experimental import pallas as pl
from jax.experimental.pallas import tpu as pltpu
from jax.experimental.pallas import tpu_sc as plsc

_VOCAB = 1000000
_HIDDEN = 64
_BATCH = 4096
_SEQ = 50
_PAD = 0
_EPS = 1e-8

_NC = 2
_NS = 16
_NW = _NC * _NS
_ROWS = _BATCH * _SEQ
_RPW = _ROWS // _NW
_PIECE = 128
_NPIECE = _RPW // _PIECE
_NBUF = 10
_GLEAD = 8

_INV_H = 1.0 / _HIDDEN


def _rsqrt(x):
    i = lax.bitcast_convert_type(x, jnp.int32)
    i = jnp.int32(0x5F3759DF) - lax.shift_right_logical(i, 1)
    y = lax.bitcast_convert_type(i, jnp.float32)
    half = x * 0.5
    y = y * (1.5 - half * y * y)
    y = y * (1.5 - half * y * y)
    y = y * (1.5 - half * y * y)
    return y


def _sc_body(tok_hbm, words_hbm, pos_hbm, wb_hbm, out_hbm,
             idx_v, pos_v, wb_v, gsems, osems, *bufs):
    wid = lax.axis_index("s") * _NC + lax.axis_index("c")
    base = wid * _RPW

    pltpu.sync_copy(tok_hbm.at[pl.ds(base, _RPW)], idx_v.at[pl.ds(0, _RPW)])
    pltpu.sync_copy(pos_hbm, pos_v)
    pltpu.sync_copy(wb_hbm, wb_v)

    def start_gather(p, buf, sem):
        pltpu.async_copy(words_hbm.at[idx_v.at[pl.ds(p * _PIECE, _PIECE)]],
                         buf, sem)

    def wait_gather(p, buf, sem):
        pltpu.make_async_copy(
            words_hbm.at[idx_v.at[pl.ds(p * _PIECE, _PIECE)]], buf, sem
        ).wait()

    def start_out(p, buf, sem):
        pltpu.async_copy(buf, out_hbm.at[pl.ds(base + p * _PIECE, _PIECE)],
                         sem)

    def wait_out(p, buf, sem):
        pltpu.make_async_copy(
            buf, out_hbm.at[pl.ds(base + p * _PIECE, _PIECE)], sem
        ).wait()

    w0 = wb_v[pl.ds(0, 16)]
    w1 = wb_v[pl.ds(16, 16)]
    w2 = wb_v[pl.ds(32, 16)]
    w3 = wb_v[pl.ds(48, 16)]
    b0 = wb_v[pl.ds(64, 16)]
    b1 = wb_v[pl.ds(80, 16)]
    b2 = wb_v[pl.ds(96, 16)]
    b3 = wb_v[pl.ds(112, 16)]

    def compute(p, rb):
        prow = p * _PIECE

        @plsc.parallel_loop(0, _PIECE, unroll=4)
        def row_body(r):
            row = prow + r
            tok = idx_v[pl.ds(row, 16)][0]
            pb = lax.rem(row, _SEQ) * _HIDDEN
            x0 = rb[r, pl.ds(0, 16)] + pos_v[pl.ds(pb, 16)]
            x1 = rb[r, pl.ds(16, 16)] + pos_v[pl.ds(pb + 16, 16)]
            x2 = rb[r, pl.ds(32, 16)] + pos_v[pl.ds(pb + 32, 16)]
            x3 = rb[r, pl.ds(48, 16)] + pos_v[pl.ds(pb + 48, 16)]
            s = (x0 + x1) + (x2 + x3)
            sq = (x0 * x0 + x1 * x1) + (x2 * x2 + x3 * x3)
            mean = plsc.cumsum(s)[15] * _INV_H
            ex2 = plsc.cumsum(sq)[15] * _INV_H
            var = ex2 - mean * mean
            rstd = _rsqrt(var + _EPS)
            scale = jnp.where(tok != _PAD, rstd, 0.0)
            shift = -mean * scale
            live = jnp.where(tok != _PAD, 1.0, 0.0)
            rb[r, pl.ds(0, 16)] = (x0 * scale + shift) * w0 + b0 * live
            rb[r, pl.ds(16, 16)] = (x1 * scale + shift) * w1 + b1 * live
            rb[r, pl.ds(32, 16)] = (x2 * scale + shift) * w2 + b2 * live
            rb[r, pl.ds(48, 16)] = (x3 * scale + shift) * w3 + b3 * live

    for j in range(_GLEAD):
        start_gather(j, bufs[j], gsems[j])

    def step(p, j):
        b = j
        wait_gather(p, bufs[b], gsems[b])
        compute(p, bufs[b])
        start_out(p, bufs[b], osems[b])
        nb = (j + _GLEAD) % _NBUF

        @pl.when(p >= _NBUF - _GLEAD)
        def _():
            wait_out(p - (_NBUF - _GLEAD), bufs[nb], osems[nb])

        @pl.when(p + _GLEAD < _NPIECE)
        def _():
            start_gather(p + _GLEAD, bufs[nb], gsems[nb])

    def main_body(g, _):
        for j in range(_NBUF):
            step(g + j, j)
        return ()

    n_main = _NPIECE // _NBUF * _NBUF
    lax.fori_loop(0, n_main // _NBUF, lambda i, c: main_body(i * _NBUF, c),
                  ())
    for p in range(n_main, _NPIECE):
        step(p, p % _NBUF)

    for p in range(_NPIECE - (_NBUF - _GLEAD), _NPIECE):
        wait_out(p, bufs[p % _NBUF], osems[p % _NBUF])


_TW = 6656


_NTB = (_VOCAB + _TW - 1) // _TW


def _tc_transpose(wT):
    def body(in_ref, out_ref):
        xt = in_ref[...].T
        out_ref[:, : _HIDDEN] = xt[: _TW // 2, :]
        out_ref[:, _HIDDEN :] = xt[_TW // 2 :, :]

    return pl.pallas_call(
        body,
        grid=(_NTB,),
        in_specs=[pl.BlockSpec((_HIDDEN, _TW), lambda i: (0, i))],
        out_specs=pl.BlockSpec((_TW // 2, 2 * _HIDDEN), lambda i: (i, 0)),
        out_shape=jax.ShapeDtypeStruct((_NTB * _TW // 2, 2 * _HIDDEN),
                                       jnp.float32),
    )(wT)


@jax.jit
def _sc_call(toks, words, pos_flat, wb):
    mesh = plsc.VectorSubcoreMesh(core_axis_name="c", subcore_axis_name="s",
                                  num_cores=_NC, num_subcores=_NS)
    f = pl.kernel(
        _sc_body,
        out_type=jax.ShapeDtypeStruct((_ROWS, _HIDDEN), jnp.float32),
        mesh=mesh,
        compiler_params=pltpu.CompilerParams(needs_layout_passes=False,
                                             use_tc_tiling_on_sc=False),
        scratch_types=[
            pltpu.VMEM((_RPW + 16,), jnp.int32),
            pltpu.VMEM((_SEQ * _HIDDEN,), jnp.float32),
            pltpu.VMEM((2 * _HIDDEN,), jnp.float32),
            [pltpu.SemaphoreType.DMA] * _NBUF,
            [pltpu.SemaphoreType.DMA] * _NBUF,
        ] + [pltpu.VMEM((_PIECE, _HIDDEN), jnp.float32)] * _NBUF,
    )
    return f(toks, words, pos_flat, wb)


def kernel(tokens, words, pos_emb, ln_weight, ln_bias):
    toks = tokens.reshape(-1).astype(jnp.int32)
    tb = toks // _TW
    tr = toks % _TW
    half = _TW // 2
    toks = tb * _TW + jnp.where(tr < half, tr * 2, (tr - half) * 2 + 1)
    wlin = _tc_transpose(words.T).reshape(_NTB * _TW, _HIDDEN)
    pos_flat = pos_emb.reshape(-1).astype(jnp.float32)
    wb = jnp.concatenate([ln_weight, ln_bias]).astype(jnp.float32)
    out = _sc_call(toks, wlin, pos_flat, wb)
    return out.reshape(tokens.shape + (_HIDDEN,))


# speedup vs baseline: 1.5393x; 1.0764x over previous
import functools

import jax
import jax.numpy as jnp
from jax import lax
from jax.experimental import pallas as pl
from jax.experimental.pallas import tpu as pltpu
from jax.experimental.pallas import tpu_sc as plsc

_VOCAB = 1000000
_HIDDEN = 64
_BATCH = 4096
_SEQ = 50
_PAD = 0
_EPS = 1e-8

_NC = 2
_NS = 16
_NW = _NC * _NS
_ROWS = _BATCH * _SEQ
_RPW = _ROWS // _NW
_PIECE = 128
_NPIECE = _RPW // _PIECE
_NBUF = 10
_GLEAD = 8

_INV_H = 1.0 / _HIDDEN


def _rsqrt(x):
    i = lax.bitcast_convert_type(x, jnp.int32)
    i = jnp.int32(0x5F3759DF) - lax.shift_right_logical(i, 1)
    y = lax.bitcast_convert_type(i, jnp.float32)
    half = x * 0.5
    y = y * (1.5 - half * y * y)
    y = y * (1.5 - half * y * y)
    y = y * (1.5 - half * y * y)
    return y


def _sc_body(tok_hbm, words_hbm, pos_hbm, wb_hbm, out_hbm,
             idx_v, pos_v, wb_v, gsems, osems, *bufs):
    wid = lax.axis_index("s") * _NC + lax.axis_index("c")
    base = wid * _RPW

    pltpu.sync_copy(tok_hbm.at[pl.ds(base, _RPW)], idx_v.at[pl.ds(0, _RPW)])
    pltpu.sync_copy(pos_hbm, pos_v)
    pltpu.sync_copy(wb_hbm, wb_v)

    def start_gather(p, buf, sem):
        pltpu.async_copy(words_hbm.at[idx_v.at[pl.ds(p * _PIECE, _PIECE)]],
                         buf, sem)

    def wait_gather(p, buf, sem):
        pltpu.make_async_copy(
            words_hbm.at[idx_v.at[pl.ds(p * _PIECE, _PIECE)]], buf, sem
        ).wait()

    def start_out(p, buf, sem):
        pltpu.async_copy(buf, out_hbm.at[pl.ds(base + p * _PIECE, _PIECE)],
                         sem)

    def wait_out(p, buf, sem):
        pltpu.make_async_copy(
            buf, out_hbm.at[pl.ds(base + p * _PIECE, _PIECE)], sem
        ).wait()

    w0 = wb_v[pl.ds(0, 16)]
    w1 = wb_v[pl.ds(16, 16)]
    w2 = wb_v[pl.ds(32, 16)]
    w3 = wb_v[pl.ds(48, 16)]
    b0 = wb_v[pl.ds(64, 16)]
    b1 = wb_v[pl.ds(80, 16)]
    b2 = wb_v[pl.ds(96, 16)]
    b3 = wb_v[pl.ds(112, 16)]

    def compute(p, rb):
        prow = p * _PIECE

        @plsc.parallel_loop(0, _PIECE, unroll=4)
        def row_body(r):
            row = prow + r
            tok = idx_v[pl.ds(row, 16)][0]
            pb = lax.rem(row, _SEQ) * _HIDDEN
            x0 = rb[r, pl.ds(0, 16)] + pos_v[pl.ds(pb, 16)]
            x1 = rb[r, pl.ds(16, 16)] + pos_v[pl.ds(pb + 16, 16)]
            x2 = rb[r, pl.ds(32, 16)] + pos_v[pl.ds(pb + 32, 16)]
            x3 = rb[r, pl.ds(48, 16)] + pos_v[pl.ds(pb + 48, 16)]
            s = (x0 + x1) + (x2 + x3)
            sq = (x0 * x0 + x1 * x1) + (x2 * x2 + x3 * x3)
            mean = plsc.cumsum(s)[15] * _INV_H
            ex2 = plsc.cumsum(sq)[15] * _INV_H
            var = ex2 - mean * mean
            rstd = _rsqrt(var + _EPS)
            scale = jnp.where(tok != _PAD, rstd, 0.0)
            shift = -mean * scale
            live = jnp.where(tok != _PAD, 1.0, 0.0)
            rb[r, pl.ds(0, 16)] = (x0 * scale + shift) * w0 + b0 * live
            rb[r, pl.ds(16, 16)] = (x1 * scale + shift) * w1 + b1 * live
            rb[r, pl.ds(32, 16)] = (x2 * scale + shift) * w2 + b2 * live
            rb[r, pl.ds(48, 16)] = (x3 * scale + shift) * w3 + b3 * live

    for j in range(_GLEAD):
        start_gather(j, bufs[j], gsems[j])

    def step(p, j):
        b = j
        wait_gather(p, bufs[b], gsems[b])
        compute(p, bufs[b])
        start_out(p, bufs[b], osems[b])
        nb = (j + _GLEAD) % _NBUF

        @pl.when(p >= _NBUF - _GLEAD)
        def _():
            wait_out(p - (_NBUF - _GLEAD), bufs[nb], osems[nb])

        @pl.when(p + _GLEAD < _NPIECE)
        def _():
            start_gather(p + _GLEAD, bufs[nb], gsems[nb])

    def main_body(g, _):
        for j in range(_NBUF):
            step(g + j, j)
        return ()

    n_main = _NPIECE // _NBUF * _NBUF
    lax.fori_loop(0, n_main // _NBUF, lambda i, c: main_body(i * _NBUF, c),
                  ())
    for p in range(n_main, _NPIECE):
        step(p, p % _NBUF)

    for p in range(_NPIECE - (_NBUF - _GLEAD), _NPIECE):
        wait_out(p, bufs[p % _NBUF], osems[p % _NBUF])


_TW = 13312


_NTB = (_VOCAB + _TW - 1) // _TW


def _tc_transpose(wT):
    def body(in_ref, out_ref):
        xt = in_ref[...].T
        out_ref[:, : _HIDDEN] = xt[: _TW // 2, :]
        out_ref[:, _HIDDEN :] = xt[_TW // 2 :, :]

    return pl.pallas_call(
        body,
        grid=(_NTB,),
        in_specs=[pl.BlockSpec((_HIDDEN, _TW), lambda i: (0, i))],
        out_specs=pl.BlockSpec((_TW // 2, 2 * _HIDDEN), lambda i: (i, 0)),
        out_shape=jax.ShapeDtypeStruct((_NTB * _TW // 2, 2 * _HIDDEN),
                                       jnp.float32),
    )(wT)


@jax.jit
def _sc_call(toks, words, pos_flat, wb):
    mesh = plsc.VectorSubcoreMesh(core_axis_name="c", subcore_axis_name="s",
                                  num_cores=_NC, num_subcores=_NS)
    f = pl.kernel(
        _sc_body,
        out_type=jax.ShapeDtypeStruct((_ROWS, _HIDDEN), jnp.float32),
        mesh=mesh,
        compiler_params=pltpu.CompilerParams(needs_layout_passes=False,
                                             use_tc_tiling_on_sc=False),
        scratch_types=[
            pltpu.VMEM((_RPW + 16,), jnp.int32),
            pltpu.VMEM((_SEQ * _HIDDEN,), jnp.float32),
            pltpu.VMEM((2 * _HIDDEN,), jnp.float32),
            [pltpu.SemaphoreType.DMA] * _NBUF,
            [pltpu.SemaphoreType.DMA] * _NBUF,
        ] + [pltpu.VMEM((_PIECE, _HIDDEN), jnp.float32)] * _NBUF,
    )
    return f(toks, words, pos_flat, wb)


def kernel(tokens, words, pos_emb, ln_weight, ln_bias):
    toks = tokens.reshape(-1).astype(jnp.int32)
    tb = toks // _TW
    tr = toks % _TW
    half = _TW // 2
    toks = tb * _TW + jnp.where(tr < half, tr * 2, (tr - half) * 2 + 1)
    wlin = _tc_transpose(words.T).reshape(_NTB * _TW, _HIDDEN)
    pos_flat = pos_emb.reshape(-1).astype(jnp.float32)
    wb = jnp.concatenate([ln_weight, ln_bias]).astype(jnp.float32)
    out = _sc_call(toks, wlin, pos_flat, wb)
    return out.reshape(tokens.shape + (_HIDDEN,))
